# baseline (speedup 1.0000x reference)
import functools

import jax
import jax.numpy as jnp
from jax import lax
from jax.experimental import pallas as pl
from jax.experimental.pallas import tpu as pltpu
from jax.experimental.pallas import tpu_sc as plsc

_CONF = 0.9


def _sc_gather(B, V, NC, bpw, out_flat, t_hbm, oh_hbm, g_out, oht_out,
               t_v, idx_v, g_v, oht_v, sem):
    wid = lax.axis_index("s") * NC + lax.axis_index("c")
    base = wid * bpw
    pltpu.sync_copy(t_hbm.at[pl.ds(base, bpw)], t_v)
    for i in range(bpw // 16):
        t16 = t_v[pl.ds(i * 16, 16)]
        rows = base + i * 16 + lax.iota(jnp.int32, 16)
        idx_v[pl.ds(i * 16, 16)] = rows * V + t16
    pltpu.async_copy(out_flat.at[idx_v], g_v, sem).wait()
    pltpu.async_copy(oh_hbm.at[t_v], oht_v, sem).wait()
    pltpu.sync_copy(g_v, g_out.at[pl.ds(base, bpw)])
    pltpu.sync_copy(oht_v, oht_out.at[pl.ds(base, bpw)])


def _tc_body(nblk, B, V, Wb, out_ref, oh_ref, g_ref, oht_ref, res_ref,
             accw_ref, acck_ref):
    k = pl.program_id(0)

    @pl.when(k == 0)
    def _init():
        accw_ref[0, 0] = 0.0
        acck_ref[0, 0] = 0.0

    x = out_ref[...]
    oh = oh_ref[...]
    col = jax.lax.broadcasted_iota(jnp.int32, (1, Wb), 1) + k * Wb
    valid = col < V

    colsum = jnp.sum(x, axis=0, keepdims=True)
    accw_ref[0, 0] += jnp.sum(jnp.where(valid, colsum * oh, 0.0))

    safe = jnp.where(oh > 0, oh, 1.0)
    kterm = jnp.where(valid & (oh > 0), oh * jnp.log(safe), 0.0)
    acck_ref[0, 0] += jnp.sum(kterm)

    @pl.when(k == nblk - 1)
    def _fin():
        g = g_ref[...]
        oht = oht_ref[...]
        safe_t = jnp.where(oht > 0, oht, 1.0)
        xlogy_t = jnp.where(oht > 0, oht * jnp.log(safe_t), 0.0)
        corr = _CONF * jnp.log(_CONF) - _CONF * g - xlogy_t + oht * g
        res_ref[0, 0] = (B * acck_ref[0, 0] - accw_ref[0, 0] + jnp.sum(corr))


def kernel(output, target, one_hot):
    B, V = output.shape
    info = plsc.get_sparse_core_info()
    NC, NS = info.num_cores, info.num_subcores
    NW = NC * NS
    bpw = B // NW

    sc = functools.partial(
        pl.kernel,
        out_type=[jax.ShapeDtypeStruct((B,), jnp.float32),
                  jax.ShapeDtypeStruct((B,), jnp.float32)],
        mesh=plsc.VectorSubcoreMesh(core_axis_name="c", subcore_axis_name="s"),
        scratch_types=[
            pltpu.VMEM((bpw,), jnp.int32),
            pltpu.VMEM((bpw,), jnp.int32),
            pltpu.VMEM((bpw,), jnp.float32),
            pltpu.VMEM((bpw,), jnp.float32),
            pltpu.SemaphoreType.DMA,
        ],
    )(functools.partial(_sc_gather, B, V, NC, bpw))
    g, oht = sc(output.reshape(B * V), target, one_hot)

    Wb = 2048
    nblk = pl.cdiv(V, Wb)
    res = pl.pallas_call(
        functools.partial(_tc_body, nblk, B, V, Wb),
        grid=(nblk,),
        in_specs=[
            pl.BlockSpec((B, Wb), lambda k: (0, k)),
            pl.BlockSpec((1, Wb), lambda k: (0, k)),
            pl.BlockSpec((B, 1), lambda k: (0, 0)),
            pl.BlockSpec((B, 1), lambda k: (0, 0)),
        ],
        out_specs=pl.BlockSpec(memory_space=pltpu.SMEM),
        out_shape=jax.ShapeDtypeStruct((1, 1), jnp.float32),
        scratch_shapes=[
            pltpu.SMEM((1, 1), jnp.float32),
            pltpu.SMEM((1, 1), jnp.float32),
        ],
        compiler_params=pltpu.CompilerParams(
            dimension_semantics=("arbitrary",),
        ),
    )(output, one_hot.reshape(1, V), g.reshape(B, 1), oht.reshape(B, 1))
    return res[0, 0]


# speedup vs baseline: 2.1785x; 2.1785x over previous
import functools

import jax
import jax.numpy as jnp
from jax.experimental import pallas as pl
from jax.experimental.pallas import tpu as pltpu

_CONF = 0.9


def _body(nblk, B, V, Wb, out_ref, t_ref, oh_ref, res_ref,
          accw_ref, acck_ref, g_ref, oht_ref):
    k = pl.program_id(0)

    @pl.when(k == 0)
    def _init():
        accw_ref[0, 0] = 0.0
        acck_ref[0, 0] = 0.0
        g_ref[...] = jnp.zeros_like(g_ref)
        oht_ref[...] = jnp.zeros_like(oht_ref)

    x = out_ref[...]
    oh = oh_ref[...]
    col = jax.lax.broadcasted_iota(jnp.int32, (1, Wb), 1) + k * Wb
    valid = col < V

    colsum = jnp.sum(x, axis=0, keepdims=True)
    accw_ref[0, 0] += jnp.sum(jnp.where(valid, colsum * oh, 0.0))

    safe = jnp.where(oh > 0, oh, 1.0)
    kterm = jnp.where(valid & (oh > 0), oh * jnp.log(safe), 0.0)
    acck_ref[0, 0] += jnp.sum(kterm)

    tcol = t_ref[...]
    cols2 = jax.lax.broadcasted_iota(jnp.int32, (B, Wb), 1) + k * Wb
    mask = cols2 == tcol
    g_ref[...] += jnp.sum(jnp.where(mask, x, 0.0), axis=1, keepdims=True)
    ohb = jnp.broadcast_to(oh, (B, Wb))
    oht_ref[...] += jnp.sum(jnp.where(mask, ohb, 0.0), axis=1, keepdims=True)

    @pl.when(k == nblk - 1)
    def _fin():
        g = g_ref[...]
        oht = oht_ref[...]
        safe_t = jnp.where(oht > 0, oht, 1.0)
        xlogy_t = jnp.where(oht > 0, oht * jnp.log(safe_t), 0.0)
        corr = _CONF * jnp.log(_CONF) - _CONF * g - xlogy_t + oht * g
        res_ref[0, 0] = (B * acck_ref[0, 0] - accw_ref[0, 0] + jnp.sum(corr))


def kernel(output, target, one_hot):
    B, V = output.shape
    Wb = 4096
    nblk = pl.cdiv(V, Wb)

    t2 = target.reshape(B, 1)
    oh2 = one_hot.reshape(1, V)

    res = pl.pallas_call(
        functools.partial(_body, nblk, B, V, Wb),
        grid=(nblk,),
        in_specs=[
            pl.BlockSpec((B, Wb), lambda k: (0, k)),
            pl.BlockSpec((B, 1), lambda k: (0, 0)),
            pl.BlockSpec((1, Wb), lambda k: (0, k)),
        ],
        out_specs=pl.BlockSpec(memory_space=pltpu.SMEM),
        out_shape=jax.ShapeDtypeStruct((1, 1), jnp.float32),
        scratch_shapes=[
            pltpu.SMEM((1, 1), jnp.float32),
            pltpu.SMEM((1, 1), jnp.float32),
            pltpu.VMEM((B, 1), jnp.float32),
            pltpu.VMEM((B, 1), jnp.float32),
        ],
        compiler_params=pltpu.CompilerParams(
            dimension_semantics=("arbitrary",),
        ),
    )(output, t2, oh2)
    return res[0, 0]
